# baseline (speedup 1.0000x reference)
import jax
import jax.numpy as jnp
from jax import lax
from jax.experimental import pallas as pl
from jax.experimental.pallas import tpu as pltpu
from jax.experimental.pallas import tpu_sc as plsc

B = 16384
VOCAB = 1000
NQ = 100
NY = 14
D = 64

NC = 2
NS = 16
L = 16
NW = NC * NS
BPW = B // NW
SLAB = 128
NSLAB = BPW // SLAB
DT = D // 8
NBC = SLAB // L
QROWS = D * SLAB // L


def _sc_body(ansT_hbm, year_hbm, table_hbm, qtab_hbm, ytab_hbm, out_hbm,
             table_v, qt_v, yt_v, yearv, ansT_v, yb_v, ob0, ob1,
             so0, so1):
    cid = lax.axis_index("c")
    sid = lax.axis_index("s")
    wid = sid * NC + cid

    out_bufs = (ob0, ob1)
    so = (so0, so1)

    pltpu.sync_copy(table_hbm, table_v)
    pltpu.sync_copy(qtab_hbm, qt_v)
    pltpu.sync_copy(ytab_hbm, yt_v)

    def wait_out(slot):
        pltpu.make_async_copy(out_bufs[slot], out_hbm.at[pl.ds(0, D)],
                              so[slot]).wait()

    for slab in range(NSLAB):
        bt = wid * NSLAB + slab
        sb = bt * SLAB

        pltpu.sync_copy(ansT_hbm.at[:, pl.ds(sb, SLAB)], ansT_v)
        pltpu.sync_copy(year_hbm.at[pl.ds(sb, SLAB)], yearv)

        def ybody(d, _):
            for bc in range(NBC):
                yid = yearv[pl.ds(16 * bc, 16)]
                yb_v[d, pl.ds(16 * bc, 16)] = plsc.load_gather(
                    yt_v, [yid, jnp.full((L,), 0, jnp.int32) + d])
            return 0

        lax.fori_loop(0, D, ybody, 0)

        for s in range(2):
            for dt in range(DT):
                base = ((s * DT + dt) * (B // SLAB) + bt) * 8
                pltpu.async_copy(out_bufs[s].at[pl.ds(dt * 8, 8)],
                                 out_hbm.at[pl.ds(base, 8)], so[s])

        def qbody(q2, _):
            for s in range(2):
                q = 2 * q2 + s
                outb = out_bufs[s]
                wait_out(s)
                aids = [ansT_v[q, pl.ds(16 * bc, 16)] for bc in range(NBC)]

                def dtbody(dt, _, outb=outb, aids=aids, q=q):
                    for din in range(8):
                        d = dt * 8 + din
                        dsplat = jnp.full((L,), 0, jnp.int32) + d
                        qsplat = plsc.load_gather(
                            qt_v, [jnp.full((L,), 0, jnp.int32) + q, dsplat])
                        for bc in range(NBC):
                            g = plsc.load_gather(table_v, [aids[bc], dsplat])
                            v = g + qsplat + yb_v[d, pl.ds(16 * bc, 16)]
                            outb[dt * 8 + din, pl.ds(16 * bc, 16)] = v
                    return 0

                lax.fori_loop(0, DT, dtbody, 0)
                for dt in range(DT):
                    base = ((q * DT + dt) * (B // SLAB) + bt) * 8
                    pltpu.async_copy(outb.at[pl.ds(dt * 8, 8)],
                                     out_hbm.at[pl.ds(base, 8)], so[s])
            return 0

        lax.fori_loop(0, NQ // 2, qbody, 0)
        for s in range(2):
            wait_out(s)


def kernel(year, answer, answer_table, year_table, question_table,
           question_range):
    year = jnp.asarray(year, jnp.int32)
    ansT = jnp.asarray(answer, jnp.int32).T
    qtab = jnp.take(question_table, question_range, axis=0)

    mesh = plsc.VectorSubcoreMesh(core_axis_name="c", subcore_axis_name="s",
                                  num_cores=NC, num_subcores=NS)
    run = pl.kernel(
        _sc_body,
        out_type=jax.ShapeDtypeStruct((NQ * D * B // 128, 128), jnp.float32),
        mesh=mesh,
        compiler_params=pltpu.CompilerParams(needs_layout_passes=False,
                                             use_tc_tiling_on_sc=False),
        scratch_types=[
            pltpu.VMEM((VOCAB, D), jnp.float32),
            pltpu.VMEM((NQ, D), jnp.float32),
            pltpu.VMEM((NY, D), jnp.float32),
            pltpu.VMEM((SLAB,), jnp.int32),
            pltpu.VMEM((NQ, SLAB), jnp.int32),
            pltpu.VMEM((D, SLAB), jnp.float32),
            pltpu.VMEM((D, SLAB), jnp.float32),
            pltpu.VMEM((D, SLAB), jnp.float32),
            pltpu.SemaphoreType.DMA,
            pltpu.SemaphoreType.DMA,
        ],
    )
    out = run(ansT, year, answer_table, qtab, year_table)
    x = out.reshape(NQ, DT, B // SLAB, 8, SLAB)
    x = x.transpose(2, 4, 0, 1, 3)
    return x.reshape(B, NQ, D)


# speedup vs baseline: 1.5780x; 1.5780x over previous
import jax
import jax.numpy as jnp
from jax import lax
from jax.experimental import pallas as pl
from jax.experimental.pallas import tpu as pltpu
from jax.experimental.pallas import tpu_sc as plsc

B = 16384
VOCAB = 1000
NQ = 100
NY = 14
D = 64

NC = 2
NS = 16
L = 16
NW = NC * NS
BPW = B // NW
SLAB = 128
NSLAB = BPW // SLAB
DT = D // 8
NBC = SLAB // L
QROWS = D * SLAB // L


def _sc_body(ansT_hbm, year_hbm, table_hbm, qtab_hbm, ytab_hbm, out_hbm,
             table_v, qt_v, yt_v, yearv, ansT_v, yb_v, ob0, ob1,
             so0, so1):
    cid = lax.axis_index("c")
    sid = lax.axis_index("s")
    wid = sid * NC + cid

    out_bufs = (ob0, ob1)
    so = (so0, so1)

    pltpu.sync_copy(table_hbm, table_v)
    pltpu.sync_copy(qtab_hbm, qt_v)
    pltpu.sync_copy(ytab_hbm, yt_v)

    def wait_out(slot):
        pltpu.make_async_copy(out_bufs[slot], out_hbm.at[pl.ds(0, D)],
                              so[slot]).wait()

    for slab in range(NSLAB):
        bt = wid * NSLAB + slab
        sb = bt * SLAB

        pltpu.sync_copy(ansT_hbm.at[:, pl.ds(sb, SLAB)], ansT_v)
        pltpu.sync_copy(year_hbm.at[pl.ds(sb, SLAB)], yearv)

        def ybody(d, _):
            for bc in range(NBC):
                yid = yearv[pl.ds(16 * bc, 16)]
                yb_v[d, pl.ds(16 * bc, 16)] = plsc.load_gather(
                    yt_v, [yid, jnp.full((L,), 0, jnp.int32) + d])
            return 0

        lax.fori_loop(0, D, ybody, 0)

        for s in range(2):
            for dt in range(DT):
                base = ((s * DT + dt) * (B // SLAB) + bt) * 8
                pltpu.async_copy(out_bufs[s].at[pl.ds(dt * 8, 8)],
                                 out_hbm.at[pl.ds(base, 8)], so[s])

        def qbody(q2, _):
            for s in range(2):
                q = 2 * q2 + s
                outb = out_bufs[s]
                wait_out(s)
                aids = [ansT_v[q, pl.ds(16 * bc, 16)] for bc in range(NBC)]

                def dtbody(dt, _, outb=outb, aids=aids, q=q):
                    for din in range(8):
                        d = dt * 8 + din
                        dsplat = jnp.full((L,), 0, jnp.int32) + d
                        qsplat = plsc.load_gather(
                            qt_v, [jnp.full((L,), 0, jnp.int32) + q, dsplat])
                        for bc in range(NBC):
                            g = plsc.load_gather(table_v, [aids[bc], dsplat])
                            v = g + qsplat + yb_v[d, pl.ds(16 * bc, 16)]
                            outb[dt * 8 + din, pl.ds(16 * bc, 16)] = v
                    return 0

                lax.fori_loop(0, DT, dtbody, 0)
                for dt in range(DT):
                    base = ((q * DT + dt) * (B // SLAB) + bt) * 8
                    pltpu.async_copy(outb.at[pl.ds(dt * 8, 8)],
                                     out_hbm.at[pl.ds(base, 8)], so[s])
            return 0

        lax.fori_loop(0, NQ // 2, qbody, 0)
        for s in range(2):
            wait_out(s)


def kernel(year, answer, answer_table, year_table, question_table,
           question_range):
    year = jnp.asarray(year, jnp.int32)
    ansT = jnp.asarray(answer, jnp.int32).T
    qtab = jnp.take(question_table, question_range, axis=0)
    tab_p = jnp.pad(answer_table, ((0, 0), (0, 1)))
    ytab_p = jnp.pad(year_table, ((0, 0), (0, 1)))

    mesh = plsc.VectorSubcoreMesh(core_axis_name="c", subcore_axis_name="s",
                                  num_cores=NC, num_subcores=NS)
    run = pl.kernel(
        _sc_body,
        out_type=jax.ShapeDtypeStruct((NQ * D * B // 128, 128), jnp.float32),
        mesh=mesh,
        compiler_params=pltpu.CompilerParams(needs_layout_passes=False,
                                             use_tc_tiling_on_sc=False),
        scratch_types=[
            pltpu.VMEM((VOCAB, D + 1), jnp.float32),
            pltpu.VMEM((NQ, D), jnp.float32),
            pltpu.VMEM((NY, D + 1), jnp.float32),
            pltpu.VMEM((SLAB,), jnp.int32),
            pltpu.VMEM((NQ, SLAB), jnp.int32),
            pltpu.VMEM((D, SLAB), jnp.float32),
            pltpu.VMEM((D, SLAB), jnp.float32),
            pltpu.VMEM((D, SLAB), jnp.float32),
            pltpu.SemaphoreType.DMA,
            pltpu.SemaphoreType.DMA,
        ],
    )
    out = run(ansT, year, tab_p, qtab, ytab_p)
    x = out.reshape(NQ, DT, B // SLAB, 8, SLAB)
    x = x.transpose(2, 4, 0, 1, 3)
    return x.reshape(B, NQ, D)


# speedup vs baseline: 3.3752x; 2.1388x over previous
import jax
import jax.numpy as jnp
from jax import lax
from jax.experimental import pallas as pl
from jax.experimental.pallas import tpu as pltpu
from jax.experimental.pallas import tpu_sc as plsc

B = 16384
VOCAB = 1000
NQ = 100
NY = 14
D = 64

NC = 2
NS = 16
L = 16
NW = NC * NS
BPW = B // NW
SLAB = 128
NSLAB = BPW // SLAB
DT = D // 8
NBC = SLAB // L
QROWS = D * SLAB // L


def _sc_body(ansT_hbm, year_hbm, table_hbm, qtab_hbm, ytab_hbm, out_hbm,
             table_v, qt_v, yt_v, yearv, ansT_v, yb_v, ob0, ob1,
             so0, so1):
    cid = lax.axis_index("c")
    sid = lax.axis_index("s")
    wid = sid * NC + cid

    out_bufs = (ob0, ob1)
    so = (so0, so1)

    pltpu.sync_copy(table_hbm, table_v)
    pltpu.sync_copy(qtab_hbm, qt_v)
    pltpu.sync_copy(ytab_hbm, yt_v)

    def wait_out(slot):
        pltpu.make_async_copy(out_bufs[slot], out_hbm.at[pl.ds(0, D)],
                              so[slot]).wait()

    for slab in range(NSLAB):
        bt = wid * NSLAB + slab
        sb = bt * SLAB

        pltpu.sync_copy(ansT_hbm.at[:, pl.ds(sb, SLAB)], ansT_v)
        pltpu.sync_copy(year_hbm.at[pl.ds(sb, SLAB)], yearv)

        def ybody(d, _):
            for bc in range(NBC):
                yid = yearv[pl.ds(16 * bc, 16)]
                yb_v[d, pl.ds(16 * bc, 16)] = plsc.load_gather(
                    yt_v, [yid, jnp.full((L,), 0, jnp.int32) + d])
            return 0

        lax.fori_loop(0, D, ybody, 0)

        for s in range(2):
            for dt in range(DT):
                base = ((s * DT + dt) * (B // SLAB) + bt) * 8
                pltpu.async_copy(out_bufs[s].at[pl.ds(dt * 8, 8)],
                                 out_hbm.at[pl.ds(base, 8)], so[s])

        def qbody(q2, _):
            for s in range(2):
                q = 2 * q2 + s
                outb = out_bufs[s]
                wait_out(s)
                aids = [ansT_v[q, pl.ds(16 * bc, 16)] for bc in range(NBC)]

                def dtbody(dt, _, outb=outb, aids=aids, q=q):
                    for din in range(8):
                        d = dt * 8 + din
                        dsplat = jnp.full((L,), 0, jnp.int32) + d
                        qsplat = plsc.load_gather(
                            qt_v, [jnp.full((L,), 0, jnp.int32) + q, dsplat])
                        gs = [plsc.load_gather(table_v, [aids[bc], dsplat])
                              for bc in range(NBC)]
                        ys = [yb_v[d, pl.ds(16 * bc, 16)] for bc in range(NBC)]
                        vs = [g + qsplat + y for g, y in zip(gs, ys)]
                        for bc in range(NBC):
                            outb[dt * 8 + din, pl.ds(16 * bc, 16)] = vs[bc]
                    return 0

                lax.fori_loop(0, DT, dtbody, 0)
                for dt in range(DT):
                    base = ((q * DT + dt) * (B // SLAB) + bt) * 8
                    pltpu.async_copy(outb.at[pl.ds(dt * 8, 8)],
                                     out_hbm.at[pl.ds(base, 8)], so[s])
            return 0

        lax.fori_loop(0, NQ // 2, qbody, 0)
        for s in range(2):
            wait_out(s)


def kernel(year, answer, answer_table, year_table, question_table,
           question_range):
    year = jnp.asarray(year, jnp.int32)
    ansT = jnp.asarray(answer, jnp.int32).T
    qtab = jnp.take(question_table, question_range, axis=0)
    tab_p = jnp.pad(answer_table, ((0, 0), (0, 1)))
    ytab_p = jnp.pad(year_table, ((0, 0), (0, 1)))

    mesh = plsc.VectorSubcoreMesh(core_axis_name="c", subcore_axis_name="s",
                                  num_cores=NC, num_subcores=NS)
    run = pl.kernel(
        _sc_body,
        out_type=jax.ShapeDtypeStruct((NQ * D * B // 128, 128), jnp.float32),
        mesh=mesh,
        compiler_params=pltpu.CompilerParams(needs_layout_passes=False,
                                             use_tc_tiling_on_sc=False),
        scratch_types=[
            pltpu.VMEM((VOCAB, D + 1), jnp.float32),
            pltpu.VMEM((NQ, D), jnp.float32),
            pltpu.VMEM((NY, D + 1), jnp.float32),
            pltpu.VMEM((SLAB,), jnp.int32),
            pltpu.VMEM((NQ, SLAB), jnp.int32),
            pltpu.VMEM((D, SLAB), jnp.float32),
            pltpu.VMEM((D, SLAB), jnp.float32),
            pltpu.VMEM((D, SLAB), jnp.float32),
            pltpu.SemaphoreType.DMA,
            pltpu.SemaphoreType.DMA,
        ],
    )
    out = run(ansT, year, tab_p, qtab, ytab_p)
    x = out.reshape(NQ, DT, B // SLAB, 8, SLAB)
    x = x.transpose(2, 4, 0, 1, 3)
    return x.reshape(B, NQ, D)


# speedup vs baseline: 4.4314x; 1.3129x over previous
import jax
import jax.numpy as jnp
from jax import lax
from jax.experimental import pallas as pl
from jax.experimental.pallas import tpu as pltpu
from jax.experimental.pallas import tpu_sc as plsc

B = 16384
VOCAB = 1000
NQ = 100
NY = 14
D = 64

NC = 2
NS = 16
L = 16
NW = NC * NS
BPW = B // NW
SLAB = 128
NSLAB = BPW // SLAB
DT = D // 8
NBC = SLAB // L
QROWS = D * SLAB // L


def _sc_body(ansT_hbm, year_hbm, table_hbm, qtab_hbm, ytab_hbm, out_hbm,
             table_v, qt_v, yt_v, yearv, ansT_v, yb_v, ob0, ob1,
             so0, so1):
    cid = lax.axis_index("c")
    sid = lax.axis_index("s")
    wid = sid * NC + cid

    out_bufs = (ob0, ob1)
    so = (so0, so1)

    pltpu.sync_copy(table_hbm, table_v)
    pltpu.sync_copy(qtab_hbm, qt_v)
    pltpu.sync_copy(ytab_hbm, yt_v)

    def wait_out(slot):
        pltpu.make_async_copy(out_bufs[slot], out_hbm.at[pl.ds(0, D)],
                              so[slot]).wait()

    for slab in range(NSLAB):
        bt = wid * NSLAB + slab
        sb = bt * SLAB

        pltpu.sync_copy(ansT_hbm.at[:, pl.ds(sb, SLAB)], ansT_v)
        pltpu.sync_copy(year_hbm.at[pl.ds(sb, SLAB)], yearv)

        def ybody(d, _):
            for bc in range(NBC):
                yid = yearv[pl.ds(16 * bc, 16)]
                yb_v[d, pl.ds(16 * bc, 16)] = plsc.load_gather(
                    yt_v, [yid, jnp.full((L,), 0, jnp.int32) + d])
            return 0

        lax.fori_loop(0, D, ybody, 0)

        for s in range(2):
            for dt in range(DT):
                base = ((s * DT + dt) * (B // SLAB) + bt) * 8
                pltpu.async_copy(out_bufs[s].at[pl.ds(dt * 8, 8)],
                                 out_hbm.at[pl.ds(base, 8)], so[s])

        def qbody(q2, _):
            for s in range(2):
                q = 2 * q2 + s
                outb = out_bufs[s]
                wait_out(s)
                def dtbody(dt, _, outb=outb, q=q):
                    base = dt * 8
                    dsplats = [jnp.full((L,), 0, jnp.int32) + (base + din)
                               for din in range(8)]
                    qrow = jnp.full((L,), 0, jnp.int32) + q
                    qs = [plsc.load_gather(qt_v, [qrow, dsplats[din]])
                          for din in range(8)]
                    for bc in range(NBC):
                        aid = ansT_v[q, pl.ds(16 * bc, 16)]
                        gs = [plsc.load_gather(table_v, [aid, dsplats[din]])
                              for din in range(8)]
                        ys = [yb_v[base + din, pl.ds(16 * bc, 16)]
                              for din in range(8)]
                        vs = [g + s + y for g, s, y in zip(gs, qs, ys)]
                        for din in range(8):
                            outb[base + din, pl.ds(16 * bc, 16)] = vs[din]
                    return 0

                lax.fori_loop(0, DT, dtbody, 0)
                for dt in range(DT):
                    base = ((q * DT + dt) * (B // SLAB) + bt) * 8
                    pltpu.async_copy(outb.at[pl.ds(dt * 8, 8)],
                                     out_hbm.at[pl.ds(base, 8)], so[s])
            return 0

        lax.fori_loop(0, NQ // 2, qbody, 0)
        for s in range(2):
            wait_out(s)


def kernel(year, answer, answer_table, year_table, question_table,
           question_range):
    year = jnp.asarray(year, jnp.int32)
    ansT = jnp.asarray(answer, jnp.int32).T
    qtab = jnp.take(question_table, question_range, axis=0)
    tab_p = jnp.pad(answer_table, ((0, 0), (0, 1)))
    ytab_p = jnp.pad(year_table, ((0, 0), (0, 1)))

    mesh = plsc.VectorSubcoreMesh(core_axis_name="c", subcore_axis_name="s",
                                  num_cores=NC, num_subcores=NS)
    run = pl.kernel(
        _sc_body,
        out_type=jax.ShapeDtypeStruct((NQ * D * B // 128, 128), jnp.float32),
        mesh=mesh,
        compiler_params=pltpu.CompilerParams(needs_layout_passes=False,
                                             use_tc_tiling_on_sc=False),
        scratch_types=[
            pltpu.VMEM((VOCAB, D + 1), jnp.float32),
            pltpu.VMEM((NQ, D), jnp.float32),
            pltpu.VMEM((NY, D + 1), jnp.float32),
            pltpu.VMEM((SLAB,), jnp.int32),
            pltpu.VMEM((NQ, SLAB), jnp.int32),
            pltpu.VMEM((D, SLAB), jnp.float32),
            pltpu.VMEM((D, SLAB), jnp.float32),
            pltpu.VMEM((D, SLAB), jnp.float32),
            pltpu.SemaphoreType.DMA,
            pltpu.SemaphoreType.DMA,
        ],
    )
    out = run(ansT, year, tab_p, qtab, ytab_p)
    x = out.reshape(NQ, DT, B // SLAB, 8, SLAB)
    x = x.transpose(2, 4, 0, 1, 3)
    return x.reshape(B, NQ, D)


# speedup vs baseline: 5.2387x; 1.1822x over previous
import jax
import jax.numpy as jnp
from jax import lax
from jax.experimental import pallas as pl
from jax.experimental.pallas import tpu as pltpu
from jax.experimental.pallas import tpu_sc as plsc

B = 16384
VOCAB = 1000
NQ = 100
NY = 14
D = 64

NC = 2
NS = 16
L = 16
NW = NC * NS
BPW = B // NW
SLAB = 128
NSLAB = BPW // SLAB
DT = D // 8
NBC = SLAB // L
QROWS = D * SLAB // L


def _lane_splat(vec, lane):
    idx = jnp.full((L, 1), lane, jnp.int32)
    dnums = lax.GatherDimensionNumbers(offset_dims=(),
                                       collapsed_slice_dims=(0,),
                                       start_index_map=(0,))
    return lax.gather(vec, idx, dnums, (1,),
                      mode=lax.GatherScatterMode.PROMISE_IN_BOUNDS)


def _sc_body(ansT_hbm, year_hbm, table_hbm, qtab_hbm, ytab_hbm, out_hbm,
             table_v, qt_v, yt_v, yearv, ansT_v, yb_v, ob0, ob1,
             so0, so1):
    cid = lax.axis_index("c")
    sid = lax.axis_index("s")
    wid = sid * NC + cid

    out_bufs = (ob0, ob1)
    so = (so0, so1)

    pltpu.sync_copy(table_hbm, table_v)
    pltpu.sync_copy(qtab_hbm, qt_v)
    pltpu.sync_copy(ytab_hbm, yt_v)

    def wait_out(slot):
        pltpu.make_async_copy(out_bufs[slot], out_hbm.at[pl.ds(0, D)],
                              so[slot]).wait()

    for slab in range(NSLAB):
        bt = wid * NSLAB + slab
        sb = bt * SLAB

        pltpu.sync_copy(ansT_hbm.at[:, pl.ds(sb, SLAB)], ansT_v)
        pltpu.sync_copy(year_hbm.at[pl.ds(sb, SLAB)], yearv)

        def ybody(d, _):
            for bc in range(NBC):
                yid = yearv[pl.ds(16 * bc, 16)]
                yb_v[d, pl.ds(16 * bc, 16)] = plsc.load_gather(
                    yt_v, [yid, jnp.full((L,), 0, jnp.int32) + d])
            return 0

        lax.fori_loop(0, D, ybody, 0)

        for s in range(2):
            for dt in range(DT):
                base = ((s * DT + dt) * (B // SLAB) + bt) * 8
                pltpu.async_copy(out_bufs[s].at[pl.ds(dt * 8, 8)],
                                 out_hbm.at[pl.ds(base, 8)], so[s])

        def qbody(q2, _):
            for s in range(2):
                q = 2 * q2 + s
                outb = out_bufs[s]
                wait_out(s)
                def dtbody(dt2, _, outb=outb, q=q):
                    base = dt2 * 16
                    dsplats = [jnp.full((L,), 0, jnp.int32) + (base + din)
                               for din in range(16)]
                    qrow = qt_v[q, pl.ds(16 * dt2, 16)]
                    qs = [_lane_splat(qrow, din) for din in range(16)]
                    for bc in range(NBC):
                        aid = ansT_v[q, pl.ds(16 * bc, 16)]
                        gs = [plsc.load_gather(table_v, [aid, dsplats[din]])
                              for din in range(16)]
                        ys = [yb_v[base + din, pl.ds(16 * bc, 16)]
                              for din in range(16)]
                        vs = [g + s + y for g, s, y in zip(gs, qs, ys)]
                        for din in range(16):
                            outb[base + din, pl.ds(16 * bc, 16)] = vs[din]
                    return 0

                lax.fori_loop(0, D // L, dtbody, 0)
                for dt in range(DT):
                    base = ((q * DT + dt) * (B // SLAB) + bt) * 8
                    pltpu.async_copy(outb.at[pl.ds(dt * 8, 8)],
                                     out_hbm.at[pl.ds(base, 8)], so[s])
            return 0

        lax.fori_loop(0, NQ // 2, qbody, 0)
        for s in range(2):
            wait_out(s)


def kernel(year, answer, answer_table, year_table, question_table,
           question_range):
    year = jnp.asarray(year, jnp.int32)
    ansT = jnp.asarray(answer, jnp.int32).T
    qtab = jnp.take(question_table, question_range, axis=0)
    tab_p = jnp.pad(answer_table, ((0, 0), (0, 1)))
    ytab_p = jnp.pad(year_table, ((0, 0), (0, 1)))

    mesh = plsc.VectorSubcoreMesh(core_axis_name="c", subcore_axis_name="s",
                                  num_cores=NC, num_subcores=NS)
    run = pl.kernel(
        _sc_body,
        out_type=jax.ShapeDtypeStruct((NQ * D * B // 128, 128), jnp.float32),
        mesh=mesh,
        compiler_params=pltpu.CompilerParams(needs_layout_passes=False,
                                             use_tc_tiling_on_sc=False),
        scratch_types=[
            pltpu.VMEM((VOCAB, D + 1), jnp.float32),
            pltpu.VMEM((NQ, D), jnp.float32),
            pltpu.VMEM((NY, D + 1), jnp.float32),
            pltpu.VMEM((SLAB,), jnp.int32),
            pltpu.VMEM((NQ, SLAB), jnp.int32),
            pltpu.VMEM((D, SLAB), jnp.float32),
            pltpu.VMEM((D, SLAB), jnp.float32),
            pltpu.VMEM((D, SLAB), jnp.float32),
            pltpu.SemaphoreType.DMA,
            pltpu.SemaphoreType.DMA,
        ],
    )
    out = run(ansT, year, tab_p, qtab, ytab_p)
    x = out.reshape(NQ, DT, B // SLAB, 8, SLAB)
    x = x.transpose(2, 4, 0, 1, 3)
    return x.reshape(B, NQ, D)


# speedup vs baseline: 5.2427x; 1.0008x over previous
import jax
import jax.numpy as jnp
from jax import lax
from jax.experimental import pallas as pl
from jax.experimental.pallas import tpu as pltpu
from jax.experimental.pallas import tpu_sc as plsc

B = 16384
VOCAB = 1000
NQ = 100
NY = 14
D = 64

NC = 2
NS = 16
L = 16
NW = NC * NS
BPW = B // NW
SLAB = 128
NSLAB = BPW // SLAB
DT = D // 8
NBC = SLAB // L


def _lane_splat(vec, lane):
    idx = jnp.full((L, 1), lane, jnp.int32)
    dnums = lax.GatherDimensionNumbers(offset_dims=(),
                                       collapsed_slice_dims=(0,),
                                       start_index_map=(0,))
    return lax.gather(vec, idx, dnums, (1,),
                      mode=lax.GatherScatterMode.PROMISE_IN_BOUNDS)


def _sc_body(ansT_hbm, year_hbm, table_hbm, qtab_hbm, ytab_hbm, out_hbm,
             table_v, qt_v, yt_v, yearv, ansT_v, yb_v, ob0, ob1,
             so0, so1):
    cid = lax.axis_index("c")
    sid = lax.axis_index("s")
    wid = sid * NC + cid

    out_bufs = (ob0, ob1)
    so = (so0, so1)

    pltpu.sync_copy(table_hbm, table_v)
    pltpu.sync_copy(qtab_hbm, qt_v)
    pltpu.sync_copy(ytab_hbm, yt_v)

    def wait_out(slot):
        pltpu.make_async_copy(out_bufs[slot], out_hbm.at[pl.ds(0, D)],
                              so[slot]).wait()

    for slab in range(NSLAB):
        bt = wid * NSLAB + slab
        sb = bt * SLAB

        pltpu.sync_copy(ansT_hbm.at[:, pl.ds(sb, SLAB)], ansT_v)
        pltpu.sync_copy(year_hbm.at[pl.ds(sb, SLAB)], yearv)

        def ybody(d, _):
            for bc in range(NBC):
                yid = yearv[pl.ds(16 * bc, 16)]
                yb_v[d, pl.ds(16 * bc, 16)] = plsc.load_gather(
                    yt_v, [yid, jnp.full((L,), 0, jnp.int32) + d])
            return 0

        lax.fori_loop(0, D, ybody, 0)

        for s in range(2):
            for dt in range(DT):
                base = ((s * DT + dt) * (B // SLAB) + bt) * 8
                pltpu.async_copy(out_bufs[s].at[pl.ds(dt * 8, 8)],
                                 out_hbm.at[pl.ds(base, 8)], so[s])

        def qbody(q2, _):
            for s in range(2):
                q = 2 * q2 + s
                outb = out_bufs[s]
                wait_out(s)
                def dtbody(dt2, _, outb=outb, q=q):
                    base = dt2 * 16
                    dsplats = [jnp.full((L,), 0, jnp.int32) + (base + din)
                               for din in range(16)]
                    qrow = qt_v[q, pl.ds(16 * dt2, 16)]
                    qs = [_lane_splat(qrow, din) for din in range(16)]
                    for bc in range(NBC):
                        aid = ansT_v[q, pl.ds(16 * bc, 16)]
                        gs = [plsc.load_gather(table_v, [aid, dsplats[din]])
                              for din in range(16)]
                        ys = [yb_v[base + din, pl.ds(16 * bc, 16)]
                              for din in range(16)]
                        vs = [g + s + y for g, s, y in zip(gs, qs, ys)]
                        for din in range(16):
                            outb[base + din, pl.ds(16 * bc, 16)] = vs[din]
                    return 0

                lax.fori_loop(0, D // L, dtbody, 0)
                for dt in range(DT):
                    base = ((q * DT + dt) * (B // SLAB) + bt) * 8
                    pltpu.async_copy(outb.at[pl.ds(dt * 8, 8)],
                                     out_hbm.at[pl.ds(base, 8)], so[s])
            return 0

        lax.fori_loop(0, NQ // 2, qbody, 0)
        for s in range(2):
            wait_out(s)


def kernel(year, answer, answer_table, year_table, question_table,
           question_range):
    year = jnp.asarray(year, jnp.int32)
    ansT = jnp.asarray(answer, jnp.int32).T
    qtab = jnp.take(question_table, question_range, axis=0)
    tab_p = jnp.pad(answer_table, ((0, 0), (0, 1)))
    ytab_p = jnp.pad(year_table, ((0, 0), (0, 1)))

    mesh = plsc.VectorSubcoreMesh(core_axis_name="c", subcore_axis_name="s",
                                  num_cores=NC, num_subcores=NS)
    run = pl.kernel(
        _sc_body,
        out_type=jax.ShapeDtypeStruct((NQ * D * B // 128, 128), jnp.float32),
        mesh=mesh,
        compiler_params=pltpu.CompilerParams(needs_layout_passes=False,
                                             use_tc_tiling_on_sc=False),
        scratch_types=[
            pltpu.VMEM((VOCAB, D + 1), jnp.float32),
            pltpu.VMEM((NQ, D), jnp.float32),
            pltpu.VMEM((NY, D + 1), jnp.float32),
            pltpu.VMEM((SLAB,), jnp.int32),
            pltpu.VMEM((NQ, SLAB), jnp.int32),
            pltpu.VMEM((D, SLAB), jnp.float32),
            pltpu.VMEM((D, SLAB), jnp.float32),
            pltpu.VMEM((D, SLAB), jnp.float32),
            pltpu.SemaphoreType.DMA,
            pltpu.SemaphoreType.DMA,
        ],
    )
    out = run(ansT, year, tab_p, qtab, ytab_p)
    x = out.reshape(NQ, DT, B // SLAB, 8, SLAB)
    x = x.transpose(2, 4, 0, 1, 3)
    return x.reshape(B, NQ, D)
